# baseline (speedup 1.0000x reference)
import functools

import jax
import jax.numpy as jnp
from jax import lax
from jax.experimental import pallas as pl
from jax.experimental.pallas import tpu as pltpu
from jax.experimental.pallas import tpu_sc as plsc

N_NODES = 10000
D = 128
E = 320000
CHUNK = 50
N_TILES = 16
NBUF = 4
CHUNKS_PER_TILE = E // (N_TILES * CHUNK)
ROWS_PER_BLK = 40
BLKS_PER_TILE = CHUNKS_PER_TILE // ROWS_PER_BLK
GROUPS_PER_BLK = ROWS_PER_BLK // NBUF
WB_CHUNK = 40
WB_CHUNKS = N_NODES // WB_CHUNK
WB_K = (WB_CHUNKS + N_TILES - 1) // N_TILES


def _msg_linears_tc(h_user, h_item, Wui, bui, Wiu, biu):
    M = h_user.shape[0]
    BM = 1000

    def body(hu_ref, hi_ref, wui_ref, bui_ref, wiu_ref, biu_ref,
             ou_ref, oi_ref):
        ou_ref[...] = lax.dot_general(
            hu_ref[...], wui_ref[...], (((1,), (1,)), ((), ())),
            preferred_element_type=jnp.float32) + bui_ref[...]
        oi_ref[...] = lax.dot_general(
            hi_ref[...], wiu_ref[...], (((1,), (1,)), ((), ())),
            preferred_element_type=jnp.float32) + biu_ref[...]

    blk = pl.BlockSpec((BM, D), lambda i: (i, 0))
    wblk = pl.BlockSpec((D, D), lambda i: (0, 0))
    bblk = pl.BlockSpec((1, D), lambda i: (0, 0))
    return pl.pallas_call(
        body,
        grid=(M // BM,),
        in_specs=[blk, blk, wblk, bblk, wblk, bblk],
        out_specs=[blk, blk],
        out_shape=[jax.ShapeDtypeStruct((M, D), jnp.float32),
                   jax.ShapeDtypeStruct((M, D), jnp.float32)],
    )(h_user, h_item, Wui, bui.reshape(1, D), Wiu, biu.reshape(1, D))


def _out_linears_tc(t_user, t_item, h_user, h_item, Au_W, Au_b, Ai_W, Ai_b,
                    skip):
    M = t_user.shape[0]
    BM = 1000

    def body(skip_ref, tu_ref, ti_ref, hu_ref, hi_ref, wu_ref, bu_ref,
             wi_ref, bi_ref, ou_ref, oi_ref):
        au = jax.nn.sigmoid(jnp.full((1, 1), skip_ref[0, 0], jnp.float32))
        ai = jax.nn.sigmoid(jnp.full((1, 1), skip_ref[0, 1], jnp.float32))
        tru = lax.dot_general(
            tu_ref[...], wu_ref[...], (((1,), (1,)), ((), ())),
            preferred_element_type=jnp.float32) + bu_ref[...]
        ou_ref[...] = tru * au + hu_ref[...] * (1.0 - au)
        tri = lax.dot_general(
            ti_ref[...], wi_ref[...], (((1,), (1,)), ((), ())),
            preferred_element_type=jnp.float32) + bi_ref[...]
        oi_ref[...] = tri * ai + hi_ref[...] * (1.0 - ai)

    blk = pl.BlockSpec((BM, D), lambda i: (i, 0))
    wblk = pl.BlockSpec((D, D), lambda i: (0, 0))
    bblk = pl.BlockSpec((1, D), lambda i: (0, 0))
    return pl.pallas_call(
        body,
        grid=(M // BM,),
        in_specs=[pl.BlockSpec(memory_space=pltpu.SMEM),
                  blk, blk, blk, blk, wblk, bblk, wblk, bblk],
        out_specs=[blk, blk],
        out_shape=[jax.ShapeDtypeStruct((M, D), jnp.float32),
                   jax.ShapeDtypeStruct((M, D), jnp.float32)],
    )(skip.reshape(1, 2), t_user, t_item, h_user, h_item,
      Au_W, Au_b.reshape(1, D), Ai_W, Ai_b.reshape(1, D))


def _sc_segment_sums(msg_ui, src_ui, dst_ui, w_ui, msg_iu, src_iu, dst_iu, w_iu):
    mesh = plsc.VectorSubcoreMesh(core_axis_name="c", subcore_axis_name="s")

    @functools.partial(
        pl.kernel,
        mesh=mesh,
        out_type=[
            jax.ShapeDtypeStruct((N_NODES, D), jnp.float32),
            jax.ShapeDtypeStruct((N_NODES, D), jnp.float32),
        ],
        scratch_types=[
            pltpu.VMEM((ROWS_PER_BLK, CHUNK), jnp.int32),
            pltpu.VMEM((ROWS_PER_BLK, CHUNK), jnp.int32),
            pltpu.VMEM((ROWS_PER_BLK * CHUNK + 16,), jnp.float32),
            [pltpu.VMEM((CHUNK, D), jnp.float32) for _ in range(NBUF)],
            pltpu.VMEM_SHARED((N_NODES, D), jnp.float32),
            [pltpu.SemaphoreType.DMA for _ in range(NBUF)],
            [pltpu.SemaphoreType.DMA for _ in range(NBUF)],
        ],
    )
    def sc_kernel(msg_ui_r, src_ui_r, dst_ui_r, w_ui_r,
                  msg_iu_r, src_iu_r, dst_iu_r, w_iu_r,
                  t_item_r, t_user_r,
                  src_blk, dst_blk, w_blk, bufs, accum, gsems, ssems):
        c = lax.axis_index("c")
        s = lax.axis_index("s")

        def zrow(r, carry):
            for q in range(D // 16):
                bufs[0][r, pl.ds(16 * q, 16)] = jnp.zeros((16,), jnp.float32)
            return carry
        lax.fori_loop(0, WB_CHUNK, zrow, 0)

        def zacc(k, carry):
            ch = s + N_TILES * k

            @pl.when(ch < WB_CHUNKS)
            def _():
                pltpu.async_copy(bufs[0].at[pl.ds(0, WB_CHUNK)],
                                 accum.at[pl.ds(ch * WB_CHUNK, WB_CHUNK)],
                                 gsems[0])
            return carry
        lax.fori_loop(0, WB_K, zacc, 0)

        def zdrain(k, carry):
            ch = s + N_TILES * k

            @pl.when(ch < WB_CHUNKS)
            def _():
                pltpu.make_async_copy(
                    bufs[0].at[pl.ds(0, WB_CHUNK)],
                    accum.at[pl.ds(ch * WB_CHUNK, WB_CHUNK)],
                    gsems[0]).wait()
            return carry
        lax.fori_loop(0, WB_K, zdrain, 0)

        plsc.subcore_barrier()

        def process(msg_r, src_r, dst_r, w_r, out_r):
            def scale_buf(buf, j):
                def escale(e, carry3):
                    wv = w_blk[pl.ds(j * CHUNK + e, 16)]
                    wspl = jnp.full((16,), wv[0], jnp.float32)
                    for q in range(D // 16):
                        sl = pl.ds(16 * q, 16)
                        buf[e, sl] = buf[e, sl] * wspl
                    return carry3
                lax.fori_loop(0, CHUNK, escale, 0, unroll=10)

            def blk_body(bk, carry):
                base = s * CHUNKS_PER_TILE + bk * ROWS_PER_BLK
                @pl.when(bk > 0)
                def _():
                    pltpu.make_async_copy(
                        bufs[2], accum.at[dst_blk.at[0]], ssems[2]).wait()
                    pltpu.make_async_copy(
                        bufs[3], accum.at[dst_blk.at[0]], ssems[3]).wait()

                pltpu.sync_copy(src_r.at[pl.ds(base, ROWS_PER_BLK)], src_blk)
                pltpu.sync_copy(dst_r.at[pl.ds(base, ROWS_PER_BLK)], dst_blk)
                pltpu.sync_copy(w_r.at[pl.ds(base * CHUNK,
                                             ROWS_PER_BLK * CHUNK)],
                                w_blk.at[pl.ds(0, ROWS_PER_BLK * CHUNK)])

                pltpu.async_copy(msg_r.at[src_blk.at[0]], bufs[0], gsems[0])
                pltpu.async_copy(msg_r.at[src_blk.at[1]], bufs[1], gsems[1])

                def group_body(g, carry2):
                    for i in range(NBUF):
                        j = NBUF * g + i
                        b = (i + 2) % NBUF
                        if i < 2:
                            @pl.when(g > 0)
                            def _():
                                pltpu.make_async_copy(
                                    bufs[b], accum.at[dst_blk.at[0]],
                                    ssems[b]).wait()

                            pltpu.async_copy(
                                msg_r.at[src_blk.at[j + 2]], bufs[b],
                                gsems[b])
                        else:
                            pltpu.make_async_copy(
                                bufs[b], accum.at[dst_blk.at[0]],
                                ssems[b]).wait()

                            @pl.when(j + 2 < ROWS_PER_BLK)
                            def _():
                                pltpu.async_copy(
                                    msg_r.at[src_blk.at[j + 2]], bufs[b],
                                    gsems[b])
                        pltpu.make_async_copy(
                            msg_r.at[src_blk.at[j]], bufs[i], gsems[i]).wait()
                        scale_buf(bufs[i], j)
                        pltpu.async_copy(bufs[i], accum.at[dst_blk.at[j]],
                                         ssems[i], add=True)
                    return carry2
                lax.fori_loop(0, GROUPS_PER_BLK, group_body, 0)
                return carry
            lax.fori_loop(0, BLKS_PER_TILE, blk_body, 0)

            pltpu.make_async_copy(
                bufs[2], accum.at[dst_blk.at[0]], ssems[2]).wait()
            pltpu.make_async_copy(
                bufs[3], accum.at[dst_blk.at[0]], ssems[3]).wait()

            plsc.subcore_barrier()

            def wb(k, carry):
                ch = s + N_TILES * k

                @pl.when(ch < WB_CHUNKS)
                def _():
                    pltpu.async_copy(
                        accum.at[pl.ds(ch * WB_CHUNK, WB_CHUNK)],
                        out_r.at[pl.ds(ch * WB_CHUNK, WB_CHUNK)], gsems[0])
                return carry
            lax.fori_loop(0, WB_K, wb, 0)

            def wbdrain(k, carry):
                ch = s + N_TILES * k

                @pl.when(ch < WB_CHUNKS)
                def _():
                    pltpu.make_async_copy(
                        accum.at[pl.ds(ch * WB_CHUNK, WB_CHUNK)],
                        out_r.at[pl.ds(ch * WB_CHUNK, WB_CHUNK)],
                        gsems[0]).wait()
                return carry
            lax.fori_loop(0, WB_K, wbdrain, 0)

        @pl.when(c == 0)
        def _():
            process(msg_ui_r, src_ui_r, dst_ui_r, w_ui_r, t_item_r)

        @pl.when(c == 1)
        def _():
            process(msg_iu_r, src_iu_r, dst_iu_r, w_iu_r, t_user_r)

    return sc_kernel(msg_ui, src_ui, dst_ui, w_ui, msg_iu, src_iu, dst_iu, w_iu)


def kernel(h_user, h_item, src_ui, dst_ui, w_ui, src_iu, dst_iu, w_iu,
           lin_ui_W, lin_ui_b, lin_iu_W, lin_iu_b,
           Au_W, Au_b, Ai_W, Ai_b, skip):
    msg_ui, msg_iu = _msg_linears_tc(h_user, h_item, lin_ui_W, lin_ui_b,
                                     lin_iu_W, lin_iu_b)
    nrows = E // CHUNK
    t_item, t_user = _sc_segment_sums(
        msg_ui, src_ui.reshape(nrows, CHUNK), dst_ui.reshape(nrows, CHUNK),
        w_ui,
        msg_iu, src_iu.reshape(nrows, CHUNK), dst_iu.reshape(nrows, CHUNK),
        w_iu)
    out_user, out_item = _out_linears_tc(
        t_user, t_item, h_user, h_item, Au_W, Au_b, Ai_W, Ai_b, skip)
    return (out_user, out_item)


# speedup vs baseline: 1.0053x; 1.0053x over previous
import functools

import jax
import jax.numpy as jnp
from jax import lax
from jax.experimental import pallas as pl
from jax.experimental.pallas import tpu as pltpu
from jax.experimental.pallas import tpu_sc as plsc

N_NODES = 10000
D = 128
E = 320000
CHUNK = 50
N_TILES = 16
NBUF = 4
CHUNKS_PER_TILE = E // (N_TILES * CHUNK)
ROWS_PER_BLK = 40
BLKS_PER_TILE = CHUNKS_PER_TILE // ROWS_PER_BLK
GROUPS_PER_BLK = ROWS_PER_BLK // NBUF
WB_CHUNK = 40
WB_CHUNKS = N_NODES // WB_CHUNK
WB_K = (WB_CHUNKS + N_TILES - 1) // N_TILES


def _msg_linears_tc(h_user, h_item, Wui, bui, Wiu, biu):
    M = h_user.shape[0]
    BM = 1000

    def body(hu_ref, hi_ref, wui_ref, bui_ref, wiu_ref, biu_ref,
             ou_ref, oi_ref):
        ou_ref[...] = lax.dot_general(
            hu_ref[...], wui_ref[...], (((1,), (1,)), ((), ())),
            preferred_element_type=jnp.float32) + bui_ref[...]
        oi_ref[...] = lax.dot_general(
            hi_ref[...], wiu_ref[...], (((1,), (1,)), ((), ())),
            preferred_element_type=jnp.float32) + biu_ref[...]

    blk = pl.BlockSpec((BM, D), lambda i: (i, 0))
    wblk = pl.BlockSpec((D, D), lambda i: (0, 0))
    bblk = pl.BlockSpec((1, D), lambda i: (0, 0))
    return pl.pallas_call(
        body,
        grid=(M // BM,),
        in_specs=[blk, blk, wblk, bblk, wblk, bblk],
        out_specs=[blk, blk],
        out_shape=[jax.ShapeDtypeStruct((M, D), jnp.float32),
                   jax.ShapeDtypeStruct((M, D), jnp.float32)],
    )(h_user, h_item, Wui, bui.reshape(1, D), Wiu, biu.reshape(1, D))


def _out_linears_tc(t_user, t_item, h_user, h_item, Au_W, Au_b, Ai_W, Ai_b,
                    skip):
    M = t_user.shape[0]
    BM = 1000

    def body(skip_ref, tu_ref, ti_ref, hu_ref, hi_ref, wu_ref, bu_ref,
             wi_ref, bi_ref, ou_ref, oi_ref):
        au = jax.nn.sigmoid(jnp.full((1, 1), skip_ref[0, 0], jnp.float32))
        ai = jax.nn.sigmoid(jnp.full((1, 1), skip_ref[0, 1], jnp.float32))
        tru = lax.dot_general(
            tu_ref[...], wu_ref[...], (((1,), (1,)), ((), ())),
            preferred_element_type=jnp.float32) + bu_ref[...]
        ou_ref[...] = tru * au + hu_ref[...] * (1.0 - au)
        tri = lax.dot_general(
            ti_ref[...], wi_ref[...], (((1,), (1,)), ((), ())),
            preferred_element_type=jnp.float32) + bi_ref[...]
        oi_ref[...] = tri * ai + hi_ref[...] * (1.0 - ai)

    blk = pl.BlockSpec((BM, D), lambda i: (i, 0))
    wblk = pl.BlockSpec((D, D), lambda i: (0, 0))
    bblk = pl.BlockSpec((1, D), lambda i: (0, 0))
    return pl.pallas_call(
        body,
        grid=(M // BM,),
        in_specs=[pl.BlockSpec(memory_space=pltpu.SMEM),
                  blk, blk, blk, blk, wblk, bblk, wblk, bblk],
        out_specs=[blk, blk],
        out_shape=[jax.ShapeDtypeStruct((M, D), jnp.float32),
                   jax.ShapeDtypeStruct((M, D), jnp.float32)],
    )(skip.reshape(1, 2), t_user, t_item, h_user, h_item,
      Au_W, Au_b.reshape(1, D), Ai_W, Ai_b.reshape(1, D))


def _sc_segment_sums(msg_ui, src_ui, dst_ui, w_ui, msg_iu, src_iu, dst_iu, w_iu):
    mesh = plsc.VectorSubcoreMesh(core_axis_name="c", subcore_axis_name="s")

    @functools.partial(
        pl.kernel,
        mesh=mesh,
        out_type=[
            jax.ShapeDtypeStruct((N_NODES, D), jnp.float32),
            jax.ShapeDtypeStruct((N_NODES, D), jnp.float32),
        ],
        scratch_types=[
            pltpu.VMEM((ROWS_PER_BLK, CHUNK), jnp.int32),
            pltpu.VMEM((ROWS_PER_BLK, CHUNK), jnp.int32),
            pltpu.VMEM((ROWS_PER_BLK * CHUNK + 16,), jnp.float32),
            [pltpu.VMEM((CHUNK, D), jnp.float32) for _ in range(NBUF)],
            pltpu.VMEM_SHARED((N_NODES, D), jnp.float32),
            [pltpu.SemaphoreType.DMA for _ in range(NBUF)],
            [pltpu.SemaphoreType.DMA for _ in range(NBUF)],
        ],
    )
    def sc_kernel(msg_ui_r, src_ui_r, dst_ui_r, w_ui_r,
                  msg_iu_r, src_iu_r, dst_iu_r, w_iu_r,
                  t_item_r, t_user_r,
                  src_blk, dst_blk, w_blk, bufs, accum, gsems, ssems):
        c = lax.axis_index("c")
        s = lax.axis_index("s")

        def zrow(r, carry):
            for q in range(D // 16):
                bufs[0][r, pl.ds(16 * q, 16)] = jnp.zeros((16,), jnp.float32)
            return carry
        lax.fori_loop(0, WB_CHUNK, zrow, 0)

        def zacc(k, carry):
            ch = s + N_TILES * k

            @pl.when(ch < WB_CHUNKS)
            def _():
                pltpu.async_copy(bufs[0].at[pl.ds(0, WB_CHUNK)],
                                 accum.at[pl.ds(ch * WB_CHUNK, WB_CHUNK)],
                                 gsems[0])
            return carry
        lax.fori_loop(0, WB_K, zacc, 0)

        def zdrain(k, carry):
            ch = s + N_TILES * k

            @pl.when(ch < WB_CHUNKS)
            def _():
                pltpu.make_async_copy(
                    bufs[0].at[pl.ds(0, WB_CHUNK)],
                    accum.at[pl.ds(ch * WB_CHUNK, WB_CHUNK)],
                    gsems[0]).wait()
            return carry
        lax.fori_loop(0, WB_K, zdrain, 0)

        plsc.subcore_barrier()

        def process(msg_r, src_r, dst_r, w_r, out_r):
            def scale_buf(buf, j):
                def escale(e, carry3):
                    wv = w_blk[pl.ds(j * CHUNK + e, 16)]
                    wspl = jnp.full((16,), wv[0], jnp.float32)
                    for q in range(D // 16):
                        sl = pl.ds(16 * q, 16)
                        buf[e, sl] = buf[e, sl] * wspl
                    return carry3
                lax.fori_loop(0, CHUNK, escale, 0, unroll=5)

            def blk_body(bk, carry):
                base = s * CHUNKS_PER_TILE + bk * ROWS_PER_BLK
                @pl.when(bk > 0)
                def _():
                    pltpu.make_async_copy(
                        bufs[2], accum.at[dst_blk.at[0]], ssems[2]).wait()
                    pltpu.make_async_copy(
                        bufs[3], accum.at[dst_blk.at[0]], ssems[3]).wait()

                pltpu.sync_copy(src_r.at[pl.ds(base, ROWS_PER_BLK)], src_blk)
                pltpu.sync_copy(dst_r.at[pl.ds(base, ROWS_PER_BLK)], dst_blk)
                pltpu.sync_copy(w_r.at[pl.ds(base * CHUNK,
                                             ROWS_PER_BLK * CHUNK)],
                                w_blk.at[pl.ds(0, ROWS_PER_BLK * CHUNK)])

                pltpu.async_copy(msg_r.at[src_blk.at[0]], bufs[0], gsems[0])
                pltpu.async_copy(msg_r.at[src_blk.at[1]], bufs[1], gsems[1])

                def group_body(g, carry2):
                    for i in range(NBUF):
                        j = NBUF * g + i
                        b = (i + 2) % NBUF
                        if i < 2:
                            @pl.when(g > 0)
                            def _():
                                pltpu.make_async_copy(
                                    bufs[b], accum.at[dst_blk.at[0]],
                                    ssems[b]).wait()

                            pltpu.async_copy(
                                msg_r.at[src_blk.at[j + 2]], bufs[b],
                                gsems[b])
                        else:
                            pltpu.make_async_copy(
                                bufs[b], accum.at[dst_blk.at[0]],
                                ssems[b]).wait()

                            @pl.when(j + 2 < ROWS_PER_BLK)
                            def _():
                                pltpu.async_copy(
                                    msg_r.at[src_blk.at[j + 2]], bufs[b],
                                    gsems[b])
                        pltpu.make_async_copy(
                            msg_r.at[src_blk.at[j]], bufs[i], gsems[i]).wait()
                        scale_buf(bufs[i], j)
                        pltpu.async_copy(bufs[i], accum.at[dst_blk.at[j]],
                                         ssems[i], add=True)
                    return carry2
                lax.fori_loop(0, GROUPS_PER_BLK, group_body, 0)
                return carry
            lax.fori_loop(0, BLKS_PER_TILE, blk_body, 0)

            pltpu.make_async_copy(
                bufs[2], accum.at[dst_blk.at[0]], ssems[2]).wait()
            pltpu.make_async_copy(
                bufs[3], accum.at[dst_blk.at[0]], ssems[3]).wait()

            plsc.subcore_barrier()

            def wb(k, carry):
                ch = s + N_TILES * k

                @pl.when(ch < WB_CHUNKS)
                def _():
                    pltpu.async_copy(
                        accum.at[pl.ds(ch * WB_CHUNK, WB_CHUNK)],
                        out_r.at[pl.ds(ch * WB_CHUNK, WB_CHUNK)], gsems[0])
                return carry
            lax.fori_loop(0, WB_K, wb, 0)

            def wbdrain(k, carry):
                ch = s + N_TILES * k

                @pl.when(ch < WB_CHUNKS)
                def _():
                    pltpu.make_async_copy(
                        accum.at[pl.ds(ch * WB_CHUNK, WB_CHUNK)],
                        out_r.at[pl.ds(ch * WB_CHUNK, WB_CHUNK)],
                        gsems[0]).wait()
                return carry
            lax.fori_loop(0, WB_K, wbdrain, 0)

        @pl.when(c == 0)
        def _():
            process(msg_ui_r, src_ui_r, dst_ui_r, w_ui_r, t_item_r)

        @pl.when(c == 1)
        def _():
            process(msg_iu_r, src_iu_r, dst_iu_r, w_iu_r, t_user_r)

    return sc_kernel(msg_ui, src_ui, dst_ui, w_ui, msg_iu, src_iu, dst_iu, w_iu)


def kernel(h_user, h_item, src_ui, dst_ui, w_ui, src_iu, dst_iu, w_iu,
           lin_ui_W, lin_ui_b, lin_iu_W, lin_iu_b,
           Au_W, Au_b, Ai_W, Ai_b, skip):
    msg_ui, msg_iu = _msg_linears_tc(h_user, h_item, lin_ui_W, lin_ui_b,
                                     lin_iu_W, lin_iu_b)
    nrows = E // CHUNK
    t_item, t_user = _sc_segment_sums(
        msg_ui, src_ui.reshape(nrows, CHUNK), dst_ui.reshape(nrows, CHUNK),
        w_ui,
        msg_iu, src_iu.reshape(nrows, CHUNK), dst_iu.reshape(nrows, CHUNK),
        w_iu)
    out_user, out_item = _out_linears_tc(
        t_user, t_item, h_user, h_item, Au_W, Au_b, Ai_W, Ai_b, skip)
    return (out_user, out_item)
